# Initial kernel scaffold; baseline (speedup 1.0000x reference)
import jax
import jax.numpy as jnp
from jax.experimental import pallas as pl


def kernel(x, edge_attr, edge_index, W, b):
    raise NotImplementedError("write your pallas kernel here")


# speedup vs baseline: 4.2558x; 4.2558x over previous
import functools

import jax
import jax.numpy as jnp
from jax import lax
from jax.experimental import pallas as pl
from jax.experimental.pallas import tpu as pltpu
from jax.experimental.pallas import tpu_sc as plsc

N_NODES = 10000
N_EDGES = 320000
D_FEAT = 128
D_EDGE = 16
D_OUT = 128

NUM_CORES = 2
NUM_SUBCORES = 16
NW = NUM_CORES * NUM_SUBCORES
E_PER_W = N_EDGES // NW
CHUNK = 40
NITER = E_PER_W // CHUNK
NBUF = 2
LANES = 16


def _proj_body(x_ref, w1_ref, w3_ref, p1_ref, p3_ref):
    xb = x_ref[...]
    p1_ref[...] = jnp.dot(xb, w1_ref[...], preferred_element_type=jnp.float32)
    p3_ref[...] = jnp.dot(xb, w3_ref[...], preferred_element_type=jnp.float32)


def _final_body(g_ref, a_ref, w2_ref, b_ref, o_ref):
    o_ref[...] = (
        g_ref[...]
        + jnp.dot(a_ref[...], w2_ref[...], preferred_element_type=jnp.float32)
        + b_ref[...]
    )


def _gather_add_body(p1_hbm, p3_hbm, src_hbm, dst_hbm, g_hbm,
                     idx_s, idx_d, rows_s, rows_d, rows_g,
                     sem_g0, sem_g1, sem_o0, sem_o1):
    sems_g = (sem_g0, sem_g1)
    sems_o = (sem_o0, sem_o1)
    wid = lax.axis_index("s") * NUM_CORES + lax.axis_index("c")

    pltpu.sync_copy(src_hbm.at[wid], idx_s)
    pltpu.sync_copy(dst_hbm.at[wid], idx_d)

    def issue_gathers(i, b):
        off = i * CHUNK
        pltpu.async_copy(
            p1_hbm.at[idx_s.at[pl.ds(off, CHUNK)]], rows_s.at[b], sems_g[b])
        pltpu.async_copy(
            p3_hbm.at[idx_d.at[pl.ds(off, CHUNK)]], rows_d.at[b], sems_g[b])

    def wait_gathers(i, b):
        off = i * CHUNK
        pltpu.make_async_copy(
            p1_hbm.at[idx_s.at[pl.ds(off, CHUNK)]], rows_s.at[b], sems_g[b]).wait()
        pltpu.make_async_copy(
            p3_hbm.at[idx_d.at[pl.ds(off, CHUNK)]], rows_d.at[b], sems_g[b]).wait()

    def out_slice(i):
        return g_hbm.at[pl.ds(wid * E_PER_W + i * CHUNK, CHUNK)]

    for b in range(NBUF):
        issue_gathers(jnp.int32(b), b)

    @pl.loop(0, NITER, step=NBUF)
    def _outer(i0):
        for b in range(NBUF):
            i = i0 + b
            wait_gathers(i, b)

            @pl.when(i + NBUF < NITER)
            def _():
                issue_gathers(i + NBUF, b)

            @pl.when(i >= NBUF)
            def _():
                pltpu.make_async_copy(rows_g.at[b], out_slice(i - NBUF),
                                      sems_o[b]).wait()

            @plsc.parallel_loop(0, CHUNK, unroll=4)
            def _add(r):
                for c in range(D_OUT // LANES):
                    sl = pl.ds(c * LANES, LANES)
                    rows_g[b, r, sl] = rows_s[b, r, sl] + rows_d[b, r, sl]

            pltpu.async_copy(rows_g.at[b], out_slice(i), sems_o[b])

    for b in range(NBUF):
        i = NITER - NBUF + b
        pltpu.make_async_copy(rows_g.at[b], out_slice(jnp.int32(i)),
                              sems_o[b]).wait()


@jax.jit
def kernel(x, edge_attr, edge_index, W, b):
    W1 = W[:D_FEAT]
    W2 = W[D_FEAT:D_FEAT + D_EDGE]
    W3 = W[D_FEAT + D_EDGE:]

    BN = 2000
    P1, P3 = pl.pallas_call(
        _proj_body,
        grid=(N_NODES // BN,),
        in_specs=[
            pl.BlockSpec((BN, D_FEAT), lambda i: (i, 0)),
            pl.BlockSpec((D_FEAT, D_OUT), lambda i: (0, 0)),
            pl.BlockSpec((D_FEAT, D_OUT), lambda i: (0, 0)),
        ],
        out_specs=[
            pl.BlockSpec((BN, D_OUT), lambda i: (i, 0)),
            pl.BlockSpec((BN, D_OUT), lambda i: (i, 0)),
        ],
        out_shape=[
            jax.ShapeDtypeStruct((N_NODES, D_OUT), jnp.float32),
            jax.ShapeDtypeStruct((N_NODES, D_OUT), jnp.float32),
        ],
    )(x, W1, W3)

    src = edge_index[0].reshape(NW, E_PER_W)
    dst = edge_index[1].reshape(NW, E_PER_W)
    mesh = plsc.VectorSubcoreMesh(core_axis_name="c", subcore_axis_name="s")
    G = pl.kernel(
        _gather_add_body,
        out_type=jax.ShapeDtypeStruct((N_EDGES, D_OUT), jnp.float32),
        mesh=mesh,
        scratch_types=[
            pltpu.VMEM((E_PER_W,), jnp.int32),
            pltpu.VMEM((E_PER_W,), jnp.int32),
            pltpu.VMEM((NBUF, CHUNK, D_OUT), jnp.float32),
            pltpu.VMEM((NBUF, CHUNK, D_OUT), jnp.float32),
            pltpu.VMEM((NBUF, CHUNK, D_OUT), jnp.float32),
            pltpu.SemaphoreType.DMA,
            pltpu.SemaphoreType.DMA,
            pltpu.SemaphoreType.DMA,
            pltpu.SemaphoreType.DMA,
        ],
    )(P1, P3, src, dst)

    BE = 4000
    b2 = b.reshape(1, D_OUT)
    out = pl.pallas_call(
        _final_body,
        grid=(N_EDGES // BE,),
        in_specs=[
            pl.BlockSpec((BE, D_OUT), lambda i: (i, 0)),
            pl.BlockSpec((BE, D_EDGE), lambda i: (i, 0)),
            pl.BlockSpec((D_EDGE, D_OUT), lambda i: (0, 0)),
            pl.BlockSpec((1, D_OUT), lambda i: (0, 0)),
        ],
        out_specs=pl.BlockSpec((BE, D_OUT), lambda i: (i, 0)),
        out_shape=jax.ShapeDtypeStruct((N_EDGES, D_OUT), jnp.float32),
    )(G, edge_attr, W2, b2)
    return out
